# Initial kernel scaffold; baseline (speedup 1.0000x reference)
import jax
import jax.numpy as jnp
from jax.experimental import pallas as pl


def kernel(solu_node, solu_eattr, a1_node, a1_eattr, a2_node, a2_eattr, b1_node, b1_eattr, b2_node, b2_eattr, facs_a, facs_b, temp_a, temp_b, params, solu_edge_index, solu_gid, a1_edge_index, a1_gid, a2_edge_index, a2_gid, b1_edge_index, b1_gid, b2_edge_index, b2_gid):
    raise NotImplementedError("write your pallas kernel here")


# speedup vs baseline: 1.9582x; 1.9582x over previous
import jax
import jax.numpy as jnp
from jax.experimental import pallas as pl
from jax.experimental.pallas import tpu as pltpu

N_NODES = 3200
N_EDGES = 6400
B = 128
D_NODE = 74
D_EDGE = 12
D_OUT = 64
D_EH = 128
EB = 800
N_EBLK = N_EDGES // EB
N_STEPS = 6
S2S_ITERS = 6


def _edge_net_kernel(eattr_ref, we1_ref, be1_ref, we2_ref, be2_ref, out_ref):
    z = jnp.dot(eattr_ref[...], we1_ref[...], preferred_element_type=jnp.float32)
    z = jnp.maximum(z + be1_ref[...], 0.0)
    ew = jnp.dot(z, we2_ref[...], preferred_element_type=jnp.float32) + be2_ref[...]
    out_ref[...] = ew.astype(jnp.bfloat16)


def _edge_net(eattr, gp):
    return pl.pallas_call(
        _edge_net_kernel,
        grid=(N_EBLK,),
        in_specs=[
            pl.BlockSpec((EB, D_EDGE), lambda j: (j, 0)),
            pl.BlockSpec((D_EDGE, D_EH), lambda j: (0, 0)),
            pl.BlockSpec((1, D_EH), lambda j: (0, 0)),
            pl.BlockSpec((D_EH, D_OUT * D_OUT), lambda j: (0, 0)),
            pl.BlockSpec((1, D_OUT * D_OUT), lambda j: (0, 0)),
        ],
        out_specs=pl.BlockSpec((EB, D_OUT * D_OUT), lambda j: (j, 0)),
        out_shape=jax.ShapeDtypeStruct((N_EDGES, D_OUT * D_OUT), jnp.bfloat16),
    )(eattr, gp['We1'], gp['be1'].reshape(1, -1), gp['We2'],
      gp['be2'].reshape(1, -1))


def _mpnn_kernel(x_ref, src_ref, dst_ref, ew_ref, wp_ref, bp_ref, bconv_ref,
                 wih_ref, whh_ref, bih_ref, bhh_ref, h_out_ref, h_ref, agg_ref):
    i = pl.program_id(0)
    j = pl.program_id(1)

    @pl.when(jnp.logical_and(i == 0, j == 0))
    def _init():
        h0 = jnp.dot(x_ref[...], wp_ref[...], preferred_element_type=jnp.float32)
        h_ref[...] = jnp.maximum(h0 + bp_ref[...], 0.0)

    iota_n = jax.lax.broadcasted_iota(jnp.int32, (N_NODES, EB), 0)
    oh_src = jnp.where(iota_n == src_ref[0], 1.0, 0.0)
    oh_dst = jnp.where(iota_n == dst_ref[0], 1.0, 0.0)

    h = h_ref[...]
    hsrc = jax.lax.dot_general(oh_src, h, (((0,), (0,)), ((), ())),
                               preferred_element_type=jnp.float32)
    ew = ew_ref[...].reshape(EB, D_OUT, D_OUT).astype(jnp.float32)
    m = jnp.sum(hsrc[:, :, None] * ew, axis=1)
    contrib = jnp.dot(oh_dst, m, preferred_element_type=jnp.float32)

    @pl.when(j == 0)
    def _first():
        agg_ref[...] = contrib

    @pl.when(j > 0)
    def _rest():
        agg_ref[...] = agg_ref[...] + contrib

    @pl.when(j == N_EBLK - 1)
    def _gru():
        a = jnp.maximum(agg_ref[...] + bconv_ref[...], 0.0)
        gi = jnp.dot(a, wih_ref[...], preferred_element_type=jnp.float32) + bih_ref[...]
        hprev = h_ref[...]
        gh = jnp.dot(hprev, whh_ref[...], preferred_element_type=jnp.float32) + bhh_ref[...]
        r = jax.nn.sigmoid(gi[:, :D_OUT] + gh[:, :D_OUT])
        z = jax.nn.sigmoid(gi[:, D_OUT:2 * D_OUT] + gh[:, D_OUT:2 * D_OUT])
        n = jnp.tanh(gi[:, 2 * D_OUT:] + r * gh[:, 2 * D_OUT:])
        hnew = (1.0 - z) * n + z * hprev
        h_ref[...] = hnew
        h_out_ref[...] = hnew


def _mpnn(x, src, dst, ew, gp):
    return pl.pallas_call(
        _mpnn_kernel,
        grid=(N_STEPS, N_EBLK),
        in_specs=[
            pl.BlockSpec((N_NODES, D_NODE), lambda i, j: (0, 0)),
            pl.BlockSpec((1, 1, EB), lambda i, j: (j, 0, 0)),
            pl.BlockSpec((1, 1, EB), lambda i, j: (j, 0, 0)),
            pl.BlockSpec((EB, D_OUT * D_OUT), lambda i, j: (j, 0)),
            pl.BlockSpec((D_NODE, D_OUT), lambda i, j: (0, 0)),
            pl.BlockSpec((1, D_OUT), lambda i, j: (0, 0)),
            pl.BlockSpec((1, D_OUT), lambda i, j: (0, 0)),
            pl.BlockSpec((D_OUT, 3 * D_OUT), lambda i, j: (0, 0)),
            pl.BlockSpec((D_OUT, 3 * D_OUT), lambda i, j: (0, 0)),
            pl.BlockSpec((1, 3 * D_OUT), lambda i, j: (0, 0)),
            pl.BlockSpec((1, 3 * D_OUT), lambda i, j: (0, 0)),
        ],
        out_specs=pl.BlockSpec((N_NODES, D_OUT), lambda i, j: (0, 0)),
        out_shape=jax.ShapeDtypeStruct((N_NODES, D_OUT), jnp.float32),
        scratch_shapes=[pltpu.VMEM((N_NODES, D_OUT), jnp.float32),
                        pltpu.VMEM((N_NODES, D_OUT), jnp.float32)],
    )(x, src, dst, ew, gp['Wp'], gp['bp'].reshape(1, -1),
      gp['bconv'].reshape(1, -1), gp['Wih'].T, gp['Whh'].T,
      gp['bih'].reshape(1, -1), gp['bhh'].reshape(1, -1))


def _s2s_kernel(feat_ref, gid_ref,
                wih0_ref, whh0_ref, bih0_ref, bhh0_ref,
                wih1_ref, whh1_ref, bih1_ref, bhh1_ref,
                wih2_ref, whh2_ref, bih2_ref, bhh2_ref, out_ref):
    feat = feat_ref[...]
    gid = gid_ref[0]
    iota_b = jax.lax.broadcasted_iota(jnp.int32, (B, N_NODES), 0)
    ohT = jnp.where(iota_b == gid, 1.0, 0.0)

    wih = [wih0_ref[...], wih1_ref[...], wih2_ref[...]]
    whh = [whh0_ref[...], whh1_ref[...], whh2_ref[...]]
    bih = [bih0_ref[...], bih1_ref[...], bih2_ref[...]]
    bhh = [bhh0_ref[...], bhh1_ref[...], bhh2_ref[...]]

    hs = [jnp.zeros((B, D_OUT), jnp.float32) for _ in range(3)]
    cs = [jnp.zeros((B, D_OUT), jnp.float32) for _ in range(3)]
    q_star = jnp.zeros((B, 2 * D_OUT), jnp.float32)

    for _ in range(S2S_ITERS):
        inp = q_star
        for l in range(3):
            gates = (jnp.dot(inp, wih[l], preferred_element_type=jnp.float32)
                     + bih[l]
                     + jnp.dot(hs[l], whh[l], preferred_element_type=jnp.float32)
                     + bhh[l])
            ii = gates[:, :D_OUT]
            ff = gates[:, D_OUT:2 * D_OUT]
            gg = gates[:, 2 * D_OUT:3 * D_OUT]
            oo = gates[:, 3 * D_OUT:]
            c = jax.nn.sigmoid(ff) * cs[l] + jax.nn.sigmoid(ii) * jnp.tanh(gg)
            hcur = jax.nn.sigmoid(oo) * jnp.tanh(c)
            hs[l] = hcur
            cs[l] = c
            inp = hcur
        q = hs[2]
        qg = jax.lax.dot_general(ohT, q, (((0,), (0,)), ((), ())),
                                 preferred_element_type=jnp.float32)
        e = jnp.sum(feat * qg, axis=1, keepdims=True)
        eT = e.reshape(1, N_NODES)
        masked = jnp.where(ohT > 0.0, eT, -1e30)
        emax = jnp.max(masked, axis=1, keepdims=True)
        emax_pn = jax.lax.dot_general(ohT, emax, (((0,), (0,)), ((), ())),
                                      preferred_element_type=jnp.float32)
        ee = jnp.exp(e - emax_pn)
        esum = jnp.dot(ohT, ee, preferred_element_type=jnp.float32)
        esum_pn = jax.lax.dot_general(ohT, esum, (((0,), (0,)), ((), ())),
                                      preferred_element_type=jnp.float32)
        alpha = ee / esum_pn
        r = jnp.dot(ohT, feat * alpha, preferred_element_type=jnp.float32)
        q_star = jnp.concatenate([q, r], axis=1)
    out_ref[...] = q_star


def _s2s(feat, gid3, sp):
    specs = [pl.BlockSpec((N_NODES, D_OUT), lambda: (0, 0)),
             pl.BlockSpec((1, 1, N_NODES), lambda: (0, 0, 0))]
    args = [feat, gid3]
    for l in ('l0', 'l1', 'l2'):
        lp = sp[l]
        in_dim = lp['Wih'].shape[1]
        specs += [pl.BlockSpec((in_dim, 4 * D_OUT), lambda: (0, 0)),
                  pl.BlockSpec((D_OUT, 4 * D_OUT), lambda: (0, 0)),
                  pl.BlockSpec((1, 4 * D_OUT), lambda: (0, 0)),
                  pl.BlockSpec((1, 4 * D_OUT), lambda: (0, 0))]
        args += [lp['Wih'].T, lp['Whh'].T,
                 lp['bih'].reshape(1, -1), lp['bhh'].reshape(1, -1)]
    return pl.pallas_call(
        _s2s_kernel,
        in_specs=specs,
        out_specs=pl.BlockSpec((B, 2 * D_OUT), lambda: (0, 0)),
        out_shape=jax.ShapeDtypeStruct((B, 2 * D_OUT), jnp.float32),
    )(*args)


def _head_kernel(qs_ref, qa1_ref, qa2_ref, qb1_ref, qb2_ref,
                 fa_ref, fb_ref, ta_ref, tb_ref,
                 w1_ref, b1_ref, w2_ref, b2_ref, out_ref):
    ga = fa_ref[:, 0:1] * qa1_ref[...] + fa_ref[:, 1:2] * qa2_ref[...]
    gb = fb_ref[:, 0:1] * qb1_ref[...] + fb_ref[:, 1:2] * qb2_ref[...]
    ta = (ta_ref[...] - 30.0) / 15.0
    tb = (tb_ref[...] - 30.0) / 15.0
    x = jnp.concatenate([qs_ref[...], ga, gb, ta, tb], axis=1)
    hmid = jnp.dot(x, w1_ref[...], preferred_element_type=jnp.float32)
    hmid = jnp.maximum(hmid + b1_ref[...], 0.0)
    out_ref[...] = (jnp.dot(hmid, w2_ref[...], preferred_element_type=jnp.float32)
                    + b2_ref[...])


def _head(qs, qa1, qa2, qb1, qb2, facs_a, facs_b, ta, tb, params):
    d_in = 2 * 3 * D_OUT + 2
    return pl.pallas_call(
        _head_kernel,
        in_specs=[pl.BlockSpec((B, 2 * D_OUT), lambda: (0, 0))] * 5
                 + [pl.BlockSpec((B, 2), lambda: (0, 0))] * 2
                 + [pl.BlockSpec((B, 1), lambda: (0, 0))] * 2
                 + [pl.BlockSpec((d_in, D_OUT), lambda: (0, 0)),
                    pl.BlockSpec((1, D_OUT), lambda: (0, 0)),
                    pl.BlockSpec((D_OUT, 1), lambda: (0, 0)),
                    pl.BlockSpec((1, 1), lambda: (0, 0))],
        out_specs=pl.BlockSpec((B, 1), lambda: (0, 0)),
        out_shape=jax.ShapeDtypeStruct((B, 1), jnp.float32),
    )(qs, qa1, qa2, qb1, qb2, facs_a, facs_b, ta, tb,
      params['W1'], params['b1'].reshape(1, -1),
      params['W2'], params['b2'].reshape(1, -1))


def kernel(solu_node, solu_eattr, a1_node, a1_eattr, a2_node, a2_eattr,
           b1_node, b1_eattr, b2_node, b2_eattr,
           facs_a, facs_b, temp_a, temp_b, params,
           solu_edge_index, solu_gid, a1_edge_index, a1_gid,
           a2_edge_index, a2_gid, b1_edge_index, b1_gid,
           b2_edge_index, b2_gid):
    p = params
    sets = [
        (solu_node, solu_eattr, solu_edge_index, solu_gid, p['gnn_solu']),
        (a1_node, a1_eattr, a1_edge_index, a1_gid, p['gnn_solv_a']),
        (a2_node, a2_eattr, a2_edge_index, a2_gid, p['gnn_solv_a']),
        (b1_node, b1_eattr, b1_edge_index, b1_gid, p['gnn_solv_b']),
        (b2_node, b2_eattr, b2_edge_index, b2_gid, p['gnn_solv_b']),
    ]
    qstars = []
    for node, eattr, ei, gid, gp in sets:
        ew = _edge_net(eattr, gp)
        src = ei[0].reshape(N_EBLK, 1, EB)
        dst = ei[1].reshape(N_EBLK, 1, EB)
        h = _mpnn(node, src, dst, ew, gp)
        qstars.append(_s2s(h, gid.reshape(1, 1, N_NODES), p['s2s']))
    return _head(qstars[0], qstars[1], qstars[2], qstars[3], qstars[4],
                 facs_a, facs_b, temp_a.reshape(-1, 1), temp_b.reshape(-1, 1),
                 p)


# speedup vs baseline: 1.9743x; 1.0082x over previous
import jax
import jax.numpy as jnp
from jax.experimental import pallas as pl
from jax.experimental.pallas import tpu as pltpu

N_NODES = 3200
N_EDGES = 6400
B = 128
D_NODE = 74
D_EDGE = 12
D_OUT = 64
D_EH = 128
EB = 800
N_EBLK = N_EDGES // EB
N_STEPS = 6
S2S_ITERS = 6


def _edge_net_kernel(eattr_ref, we1_ref, be1_ref, we2_ref, be2_ref, out_ref):
    z = jnp.dot(eattr_ref[...], we1_ref[...], preferred_element_type=jnp.float32)
    z = jnp.maximum(z + be1_ref[...], 0.0)
    ew = jnp.dot(z, we2_ref[...], preferred_element_type=jnp.float32) + be2_ref[...]
    out_ref[...] = ew.astype(jnp.bfloat16)


def _edge_net(eattr, gp):
    return pl.pallas_call(
        _edge_net_kernel,
        grid=(N_EBLK,),
        in_specs=[
            pl.BlockSpec((EB, D_EDGE), lambda j: (j, 0)),
            pl.BlockSpec((D_EDGE, D_EH), lambda j: (0, 0)),
            pl.BlockSpec((1, D_EH), lambda j: (0, 0)),
            pl.BlockSpec((D_EH, D_OUT * D_OUT), lambda j: (0, 0)),
            pl.BlockSpec((1, D_OUT * D_OUT), lambda j: (0, 0)),
        ],
        out_specs=pl.BlockSpec((EB, D_OUT * D_OUT), lambda j: (j, 0)),
        out_shape=jax.ShapeDtypeStruct((N_EDGES, D_OUT * D_OUT), jnp.bfloat16),
    )(eattr, gp['We1'], gp['be1'].reshape(1, -1), gp['We2'],
      gp['be2'].reshape(1, -1))


def _mpnn_kernel(x_ref, src_ref, dst_ref, ew_ref, wp_ref, bp_ref, bconv_ref,
                 wih_ref, whh_ref, bih_ref, bhh_ref, h_out_ref, h_ref, agg_ref):
    i = pl.program_id(0)
    j = pl.program_id(1)

    @pl.when(jnp.logical_and(i == 0, j == 0))
    def _init():
        h0 = jnp.dot(x_ref[...], wp_ref[...], preferred_element_type=jnp.float32)
        h_ref[...] = jnp.maximum(h0 + bp_ref[...], 0.0)

    iota_n = jax.lax.broadcasted_iota(jnp.int32, (N_NODES, EB), 0)
    oh_src = jnp.where(iota_n == src_ref[0], 1.0, 0.0).astype(jnp.bfloat16)
    oh_dst = jnp.where(iota_n == dst_ref[0], 1.0, 0.0).astype(jnp.bfloat16)

    h = h_ref[...].astype(jnp.bfloat16)
    hsrc = jax.lax.dot_general(oh_src, h, (((0,), (0,)), ((), ())),
                               preferred_element_type=jnp.float32
                               ).astype(jnp.bfloat16)
    ew = ew_ref[...].reshape(EB, D_OUT, D_OUT)
    m = jnp.sum(hsrc[:, :, None] * ew, axis=1,
                dtype=jnp.float32).astype(jnp.bfloat16)
    contrib = jnp.dot(oh_dst, m, preferred_element_type=jnp.float32)

    @pl.when(j == 0)
    def _first():
        agg_ref[...] = contrib

    @pl.when(j > 0)
    def _rest():
        agg_ref[...] = agg_ref[...] + contrib

    @pl.when(j == N_EBLK - 1)
    def _gru():
        a = jnp.maximum(agg_ref[...] + bconv_ref[...], 0.0)
        gi = jnp.dot(a, wih_ref[...], preferred_element_type=jnp.float32) + bih_ref[...]
        hprev = h_ref[...]
        gh = jnp.dot(hprev, whh_ref[...], preferred_element_type=jnp.float32) + bhh_ref[...]
        r = jax.nn.sigmoid(gi[:, :D_OUT] + gh[:, :D_OUT])
        z = jax.nn.sigmoid(gi[:, D_OUT:2 * D_OUT] + gh[:, D_OUT:2 * D_OUT])
        n = jnp.tanh(gi[:, 2 * D_OUT:] + r * gh[:, 2 * D_OUT:])
        hnew = (1.0 - z) * n + z * hprev
        h_ref[...] = hnew
        h_out_ref[...] = hnew


def _mpnn(x, src, dst, ew, gp):
    return pl.pallas_call(
        _mpnn_kernel,
        grid=(N_STEPS, N_EBLK),
        in_specs=[
            pl.BlockSpec((N_NODES, D_NODE), lambda i, j: (0, 0)),
            pl.BlockSpec((1, 1, EB), lambda i, j: (j, 0, 0)),
            pl.BlockSpec((1, 1, EB), lambda i, j: (j, 0, 0)),
            pl.BlockSpec((EB, D_OUT * D_OUT), lambda i, j: (j, 0)),
            pl.BlockSpec((D_NODE, D_OUT), lambda i, j: (0, 0)),
            pl.BlockSpec((1, D_OUT), lambda i, j: (0, 0)),
            pl.BlockSpec((1, D_OUT), lambda i, j: (0, 0)),
            pl.BlockSpec((D_OUT, 3 * D_OUT), lambda i, j: (0, 0)),
            pl.BlockSpec((D_OUT, 3 * D_OUT), lambda i, j: (0, 0)),
            pl.BlockSpec((1, 3 * D_OUT), lambda i, j: (0, 0)),
            pl.BlockSpec((1, 3 * D_OUT), lambda i, j: (0, 0)),
        ],
        out_specs=pl.BlockSpec((N_NODES, D_OUT), lambda i, j: (0, 0)),
        out_shape=jax.ShapeDtypeStruct((N_NODES, D_OUT), jnp.float32),
        scratch_shapes=[pltpu.VMEM((N_NODES, D_OUT), jnp.float32),
                        pltpu.VMEM((N_NODES, D_OUT), jnp.float32)],
    )(x, src, dst, ew, gp['Wp'], gp['bp'].reshape(1, -1),
      gp['bconv'].reshape(1, -1), gp['Wih'].T, gp['Whh'].T,
      gp['bih'].reshape(1, -1), gp['bhh'].reshape(1, -1))


def _s2s_kernel(feat_ref, gid_ref,
                wih0_ref, whh0_ref, bih0_ref, bhh0_ref,
                wih1_ref, whh1_ref, bih1_ref, bhh1_ref,
                wih2_ref, whh2_ref, bih2_ref, bhh2_ref, out_ref):
    feat = feat_ref[...]
    gid = gid_ref[0]
    iota_b = jax.lax.broadcasted_iota(jnp.int32, (B, N_NODES), 0)
    ohT = jnp.where(iota_b == gid, 1.0, 0.0)

    wih = [wih0_ref[...], wih1_ref[...], wih2_ref[...]]
    whh = [whh0_ref[...], whh1_ref[...], whh2_ref[...]]
    bih = [bih0_ref[...], bih1_ref[...], bih2_ref[...]]
    bhh = [bhh0_ref[...], bhh1_ref[...], bhh2_ref[...]]

    hs = [jnp.zeros((B, D_OUT), jnp.float32) for _ in range(3)]
    cs = [jnp.zeros((B, D_OUT), jnp.float32) for _ in range(3)]
    q_star = jnp.zeros((B, 2 * D_OUT), jnp.float32)

    for _ in range(S2S_ITERS):
        inp = q_star
        for l in range(3):
            gates = (jnp.dot(inp, wih[l], preferred_element_type=jnp.float32)
                     + bih[l]
                     + jnp.dot(hs[l], whh[l], preferred_element_type=jnp.float32)
                     + bhh[l])
            ii = gates[:, :D_OUT]
            ff = gates[:, D_OUT:2 * D_OUT]
            gg = gates[:, 2 * D_OUT:3 * D_OUT]
            oo = gates[:, 3 * D_OUT:]
            c = jax.nn.sigmoid(ff) * cs[l] + jax.nn.sigmoid(ii) * jnp.tanh(gg)
            hcur = jax.nn.sigmoid(oo) * jnp.tanh(c)
            hs[l] = hcur
            cs[l] = c
            inp = hcur
        q = hs[2]
        qg = jax.lax.dot_general(ohT, q, (((0,), (0,)), ((), ())),
                                 preferred_element_type=jnp.float32)
        e = jnp.sum(feat * qg, axis=1, keepdims=True)
        eT = e.reshape(1, N_NODES)
        masked = jnp.where(ohT > 0.0, eT, -1e30)
        emax = jnp.max(masked, axis=1, keepdims=True)
        emax_pn = jax.lax.dot_general(ohT, emax, (((0,), (0,)), ((), ())),
                                      preferred_element_type=jnp.float32)
        ee = jnp.exp(e - emax_pn)
        esum = jnp.dot(ohT, ee, preferred_element_type=jnp.float32)
        esum_pn = jax.lax.dot_general(ohT, esum, (((0,), (0,)), ((), ())),
                                      preferred_element_type=jnp.float32)
        alpha = ee / esum_pn
        r = jnp.dot(ohT, feat * alpha, preferred_element_type=jnp.float32)
        q_star = jnp.concatenate([q, r], axis=1)
    out_ref[...] = q_star


def _s2s(feat, gid3, sp):
    specs = [pl.BlockSpec((N_NODES, D_OUT), lambda: (0, 0)),
             pl.BlockSpec((1, 1, N_NODES), lambda: (0, 0, 0))]
    args = [feat, gid3]
    for l in ('l0', 'l1', 'l2'):
        lp = sp[l]
        in_dim = lp['Wih'].shape[1]
        specs += [pl.BlockSpec((in_dim, 4 * D_OUT), lambda: (0, 0)),
                  pl.BlockSpec((D_OUT, 4 * D_OUT), lambda: (0, 0)),
                  pl.BlockSpec((1, 4 * D_OUT), lambda: (0, 0)),
                  pl.BlockSpec((1, 4 * D_OUT), lambda: (0, 0))]
        args += [lp['Wih'].T, lp['Whh'].T,
                 lp['bih'].reshape(1, -1), lp['bhh'].reshape(1, -1)]
    return pl.pallas_call(
        _s2s_kernel,
        in_specs=specs,
        out_specs=pl.BlockSpec((B, 2 * D_OUT), lambda: (0, 0)),
        out_shape=jax.ShapeDtypeStruct((B, 2 * D_OUT), jnp.float32),
    )(*args)


def _head_kernel(qs_ref, qa1_ref, qa2_ref, qb1_ref, qb2_ref,
                 fa_ref, fb_ref, ta_ref, tb_ref,
                 w1_ref, b1_ref, w2_ref, b2_ref, out_ref):
    ga = fa_ref[:, 0:1] * qa1_ref[...] + fa_ref[:, 1:2] * qa2_ref[...]
    gb = fb_ref[:, 0:1] * qb1_ref[...] + fb_ref[:, 1:2] * qb2_ref[...]
    ta = (ta_ref[...] - 30.0) / 15.0
    tb = (tb_ref[...] - 30.0) / 15.0
    x = jnp.concatenate([qs_ref[...], ga, gb, ta, tb], axis=1)
    hmid = jnp.dot(x, w1_ref[...], preferred_element_type=jnp.float32)
    hmid = jnp.maximum(hmid + b1_ref[...], 0.0)
    out_ref[...] = (jnp.dot(hmid, w2_ref[...], preferred_element_type=jnp.float32)
                    + b2_ref[...])


def _head(qs, qa1, qa2, qb1, qb2, facs_a, facs_b, ta, tb, params):
    d_in = 2 * 3 * D_OUT + 2
    return pl.pallas_call(
        _head_kernel,
        in_specs=[pl.BlockSpec((B, 2 * D_OUT), lambda: (0, 0))] * 5
                 + [pl.BlockSpec((B, 2), lambda: (0, 0))] * 2
                 + [pl.BlockSpec((B, 1), lambda: (0, 0))] * 2
                 + [pl.BlockSpec((d_in, D_OUT), lambda: (0, 0)),
                    pl.BlockSpec((1, D_OUT), lambda: (0, 0)),
                    pl.BlockSpec((D_OUT, 1), lambda: (0, 0)),
                    pl.BlockSpec((1, 1), lambda: (0, 0))],
        out_specs=pl.BlockSpec((B, 1), lambda: (0, 0)),
        out_shape=jax.ShapeDtypeStruct((B, 1), jnp.float32),
    )(qs, qa1, qa2, qb1, qb2, facs_a, facs_b, ta, tb,
      params['W1'], params['b1'].reshape(1, -1),
      params['W2'], params['b2'].reshape(1, -1))


def kernel(solu_node, solu_eattr, a1_node, a1_eattr, a2_node, a2_eattr,
           b1_node, b1_eattr, b2_node, b2_eattr,
           facs_a, facs_b, temp_a, temp_b, params,
           solu_edge_index, solu_gid, a1_edge_index, a1_gid,
           a2_edge_index, a2_gid, b1_edge_index, b1_gid,
           b2_edge_index, b2_gid):
    p = params
    sets = [
        (solu_node, solu_eattr, solu_edge_index, solu_gid, p['gnn_solu']),
        (a1_node, a1_eattr, a1_edge_index, a1_gid, p['gnn_solv_a']),
        (a2_node, a2_eattr, a2_edge_index, a2_gid, p['gnn_solv_a']),
        (b1_node, b1_eattr, b1_edge_index, b1_gid, p['gnn_solv_b']),
        (b2_node, b2_eattr, b2_edge_index, b2_gid, p['gnn_solv_b']),
    ]
    qstars = []
    for node, eattr, ei, gid, gp in sets:
        ew = _edge_net(eattr, gp)
        src = ei[0].reshape(N_EBLK, 1, EB)
        dst = ei[1].reshape(N_EBLK, 1, EB)
        h = _mpnn(node, src, dst, ew, gp)
        qstars.append(_s2s(h, gid.reshape(1, 1, N_NODES), p['s2s']))
    return _head(qstars[0], qstars[1], qstars[2], qstars[3], qstars[4],
                 facs_a, facs_b, temp_a.reshape(-1, 1), temp_b.reshape(-1, 1),
                 p)


# speedup vs baseline: 2.8845x; 1.4610x over previous
import jax
import jax.numpy as jnp
from jax.experimental import pallas as pl
from jax.experimental.pallas import tpu as pltpu

N_NODES = 3200
N_EDGES = 6400
B = 128
D_NODE = 74
D_EDGE = 12
D_OUT = 64
D_EH = 128
EB = 800
N_EBLK = N_EDGES // EB
N_STEPS = 6
S2S_ITERS = 6


def _edge_net_kernel(eattr_ref, we1_ref, be1_ref, we2_ref, be2_ref, out_ref):
    z = jnp.dot(eattr_ref[...], we1_ref[...], preferred_element_type=jnp.float32)
    z = jnp.maximum(z + be1_ref[...], 0.0)
    ew = jnp.dot(z, we2_ref[...], preferred_element_type=jnp.float32) + be2_ref[...]
    out_ref[...] = ew.astype(jnp.bfloat16)


def _edge_net(eattr, gp):
    return pl.pallas_call(
        _edge_net_kernel,
        grid=(N_EBLK,),
        in_specs=[
            pl.BlockSpec((EB, D_EDGE), lambda j: (j, 0)),
            pl.BlockSpec((D_EDGE, D_EH), lambda j: (0, 0)),
            pl.BlockSpec((1, D_EH), lambda j: (0, 0)),
            pl.BlockSpec((D_EH, D_OUT * D_OUT), lambda j: (0, 0)),
            pl.BlockSpec((1, D_OUT * D_OUT), lambda j: (0, 0)),
        ],
        out_specs=pl.BlockSpec((EB, D_OUT * D_OUT), lambda j: (j, 0)),
        out_shape=jax.ShapeDtypeStruct((N_EDGES, D_OUT * D_OUT), jnp.bfloat16),
    )(eattr, gp['We1'], gp['be1'].reshape(1, -1), gp['We2'],
      gp['be2'].reshape(1, -1))


def _tile_mat():
    c = jnp.arange(D_OUT * D_OUT, dtype=jnp.int32) // D_OUT
    return (c[None, :] == jnp.arange(D_OUT, dtype=jnp.int32)[:, None]
            ).astype(jnp.float32)


def _mpnn_kernel(x_ref, src_ref, dst_ref, ew_ref, r_ref, wp_ref, bp_ref,
                 bconv_ref, wih_ref, whh_ref, bih_ref, bhh_ref,
                 h_out_ref, h_ref, agg_ref):
    i = pl.program_id(0)
    j = pl.program_id(1)

    @pl.when(jnp.logical_and(i == 0, j == 0))
    def _init():
        h0 = jnp.dot(x_ref[...], wp_ref[...], preferred_element_type=jnp.float32)
        h_ref[...] = jnp.maximum(h0 + bp_ref[...], 0.0)

    iota_n = jax.lax.broadcasted_iota(jnp.int32, (N_NODES, EB), 0)
    oh_src = jnp.where(iota_n == src_ref[0], 1.0, 0.0)
    oh_dst = jnp.where(iota_n == dst_ref[0], 1.0, 0.0)

    h = h_ref[...]
    hsrc = jax.lax.dot_general(oh_src, h, (((0,), (0,)), ((), ())),
                               preferred_element_type=jnp.float32)
    hsrc_t = jnp.dot(hsrc, r_ref[...], preferred_element_type=jnp.float32)
    q = hsrc_t * ew_ref[...].astype(jnp.float32)
    w = D_OUT * D_OUT // 2
    while w >= D_OUT:
        q = q[:, :w] + q[:, w:2 * w]
        w //= 2
    m = q
    contrib = jnp.dot(oh_dst, m, preferred_element_type=jnp.float32)

    @pl.when(j == 0)
    def _first():
        agg_ref[...] = contrib

    @pl.when(j > 0)
    def _rest():
        agg_ref[...] = agg_ref[...] + contrib

    @pl.when(j == N_EBLK - 1)
    def _gru():
        a = jnp.maximum(agg_ref[...] + bconv_ref[...], 0.0)
        gi = jnp.dot(a, wih_ref[...], preferred_element_type=jnp.float32) + bih_ref[...]
        hprev = h_ref[...]
        gh = jnp.dot(hprev, whh_ref[...], preferred_element_type=jnp.float32) + bhh_ref[...]
        r = jax.nn.sigmoid(gi[:, :D_OUT] + gh[:, :D_OUT])
        z = jax.nn.sigmoid(gi[:, D_OUT:2 * D_OUT] + gh[:, D_OUT:2 * D_OUT])
        n = jnp.tanh(gi[:, 2 * D_OUT:] + r * gh[:, 2 * D_OUT:])
        hnew = (1.0 - z) * n + z * hprev
        h_ref[...] = hnew
        h_out_ref[...] = hnew


def _mpnn(x, src, dst, ew, gp):
    return pl.pallas_call(
        _mpnn_kernel,
        grid=(N_STEPS, N_EBLK),
        in_specs=[
            pl.BlockSpec((N_NODES, D_NODE), lambda i, j: (0, 0)),
            pl.BlockSpec((1, 1, EB), lambda i, j: (j, 0, 0)),
            pl.BlockSpec((1, 1, EB), lambda i, j: (j, 0, 0)),
            pl.BlockSpec((EB, D_OUT * D_OUT), lambda i, j: (j, 0)),
            pl.BlockSpec((D_OUT, D_OUT * D_OUT), lambda i, j: (0, 0)),
            pl.BlockSpec((D_NODE, D_OUT), lambda i, j: (0, 0)),
            pl.BlockSpec((1, D_OUT), lambda i, j: (0, 0)),
            pl.BlockSpec((1, D_OUT), lambda i, j: (0, 0)),
            pl.BlockSpec((D_OUT, 3 * D_OUT), lambda i, j: (0, 0)),
            pl.BlockSpec((D_OUT, 3 * D_OUT), lambda i, j: (0, 0)),
            pl.BlockSpec((1, 3 * D_OUT), lambda i, j: (0, 0)),
            pl.BlockSpec((1, 3 * D_OUT), lambda i, j: (0, 0)),
        ],
        out_specs=pl.BlockSpec((N_NODES, D_OUT), lambda i, j: (0, 0)),
        out_shape=jax.ShapeDtypeStruct((N_NODES, D_OUT), jnp.float32),
        scratch_shapes=[pltpu.VMEM((N_NODES, D_OUT), jnp.float32),
                        pltpu.VMEM((N_NODES, D_OUT), jnp.float32)],
    )(x, src, dst, ew, _tile_mat(), gp['Wp'], gp['bp'].reshape(1, -1),
      gp['bconv'].reshape(1, -1), gp['Wih'].T, gp['Whh'].T,
      gp['bih'].reshape(1, -1), gp['bhh'].reshape(1, -1))


def _s2s_kernel(feat_ref, gid_ref,
                wih0_ref, whh0_ref, bih0_ref, bhh0_ref,
                wih1_ref, whh1_ref, bih1_ref, bhh1_ref,
                wih2_ref, whh2_ref, bih2_ref, bhh2_ref, out_ref):
    feat = feat_ref[...]
    gid = gid_ref[0]
    iota_b = jax.lax.broadcasted_iota(jnp.int32, (B, N_NODES), 0)
    ohT = jnp.where(iota_b == gid, 1.0, 0.0)

    wih = [wih0_ref[...], wih1_ref[...], wih2_ref[...]]
    whh = [whh0_ref[...], whh1_ref[...], whh2_ref[...]]
    bih = [bih0_ref[...], bih1_ref[...], bih2_ref[...]]
    bhh = [bhh0_ref[...], bhh1_ref[...], bhh2_ref[...]]

    hs = [jnp.zeros((B, D_OUT), jnp.float32) for _ in range(3)]
    cs = [jnp.zeros((B, D_OUT), jnp.float32) for _ in range(3)]
    q_star = jnp.zeros((B, 2 * D_OUT), jnp.float32)

    for _ in range(S2S_ITERS):
        inp = q_star
        for l in range(3):
            gates = (jnp.dot(inp, wih[l], preferred_element_type=jnp.float32)
                     + bih[l]
                     + jnp.dot(hs[l], whh[l], preferred_element_type=jnp.float32)
                     + bhh[l])
            ii = gates[:, :D_OUT]
            ff = gates[:, D_OUT:2 * D_OUT]
            gg = gates[:, 2 * D_OUT:3 * D_OUT]
            oo = gates[:, 3 * D_OUT:]
            c = jax.nn.sigmoid(ff) * cs[l] + jax.nn.sigmoid(ii) * jnp.tanh(gg)
            hcur = jax.nn.sigmoid(oo) * jnp.tanh(c)
            hs[l] = hcur
            cs[l] = c
            inp = hcur
        q = hs[2]
        qg = jax.lax.dot_general(ohT, q, (((0,), (0,)), ((), ())),
                                 preferred_element_type=jnp.float32)
        e = jnp.sum(feat * qg, axis=1, keepdims=True)
        eT = e.reshape(1, N_NODES)
        masked = jnp.where(ohT > 0.0, eT, -1e30)
        emax = jnp.max(masked, axis=1, keepdims=True)
        emax_pn = jax.lax.dot_general(ohT, emax, (((0,), (0,)), ((), ())),
                                      preferred_element_type=jnp.float32)
        ee = jnp.exp(e - emax_pn)
        esum = jnp.dot(ohT, ee, preferred_element_type=jnp.float32)
        esum_pn = jax.lax.dot_general(ohT, esum, (((0,), (0,)), ((), ())),
                                      preferred_element_type=jnp.float32)
        alpha = ee / esum_pn
        r = jnp.dot(ohT, feat * alpha, preferred_element_type=jnp.float32)
        q_star = jnp.concatenate([q, r], axis=1)
    out_ref[...] = q_star


def _s2s(feat, gid3, sp):
    specs = [pl.BlockSpec((N_NODES, D_OUT), lambda: (0, 0)),
             pl.BlockSpec((1, 1, N_NODES), lambda: (0, 0, 0))]
    args = [feat, gid3]
    for l in ('l0', 'l1', 'l2'):
        lp = sp[l]
        in_dim = lp['Wih'].shape[1]
        specs += [pl.BlockSpec((in_dim, 4 * D_OUT), lambda: (0, 0)),
                  pl.BlockSpec((D_OUT, 4 * D_OUT), lambda: (0, 0)),
                  pl.BlockSpec((1, 4 * D_OUT), lambda: (0, 0)),
                  pl.BlockSpec((1, 4 * D_OUT), lambda: (0, 0))]
        args += [lp['Wih'].T, lp['Whh'].T,
                 lp['bih'].reshape(1, -1), lp['bhh'].reshape(1, -1)]
    return pl.pallas_call(
        _s2s_kernel,
        in_specs=specs,
        out_specs=pl.BlockSpec((B, 2 * D_OUT), lambda: (0, 0)),
        out_shape=jax.ShapeDtypeStruct((B, 2 * D_OUT), jnp.float32),
    )(*args)


def _head_kernel(qs_ref, qa1_ref, qa2_ref, qb1_ref, qb2_ref,
                 fa_ref, fb_ref, ta_ref, tb_ref,
                 w1_ref, b1_ref, w2_ref, b2_ref, out_ref):
    ga = fa_ref[:, 0:1] * qa1_ref[...] + fa_ref[:, 1:2] * qa2_ref[...]
    gb = fb_ref[:, 0:1] * qb1_ref[...] + fb_ref[:, 1:2] * qb2_ref[...]
    ta = (ta_ref[...] - 30.0) / 15.0
    tb = (tb_ref[...] - 30.0) / 15.0
    x = jnp.concatenate([qs_ref[...], ga, gb, ta, tb], axis=1)
    hmid = jnp.dot(x, w1_ref[...], preferred_element_type=jnp.float32)
    hmid = jnp.maximum(hmid + b1_ref[...], 0.0)
    out_ref[...] = (jnp.dot(hmid, w2_ref[...], preferred_element_type=jnp.float32)
                    + b2_ref[...])


def _head(qs, qa1, qa2, qb1, qb2, facs_a, facs_b, ta, tb, params):
    d_in = 2 * 3 * D_OUT + 2
    return pl.pallas_call(
        _head_kernel,
        in_specs=[pl.BlockSpec((B, 2 * D_OUT), lambda: (0, 0))] * 5
                 + [pl.BlockSpec((B, 2), lambda: (0, 0))] * 2
                 + [pl.BlockSpec((B, 1), lambda: (0, 0))] * 2
                 + [pl.BlockSpec((d_in, D_OUT), lambda: (0, 0)),
                    pl.BlockSpec((1, D_OUT), lambda: (0, 0)),
                    pl.BlockSpec((D_OUT, 1), lambda: (0, 0)),
                    pl.BlockSpec((1, 1), lambda: (0, 0))],
        out_specs=pl.BlockSpec((B, 1), lambda: (0, 0)),
        out_shape=jax.ShapeDtypeStruct((B, 1), jnp.float32),
    )(qs, qa1, qa2, qb1, qb2, facs_a, facs_b, ta, tb,
      params['W1'], params['b1'].reshape(1, -1),
      params['W2'], params['b2'].reshape(1, -1))


def kernel(solu_node, solu_eattr, a1_node, a1_eattr, a2_node, a2_eattr,
           b1_node, b1_eattr, b2_node, b2_eattr,
           facs_a, facs_b, temp_a, temp_b, params,
           solu_edge_index, solu_gid, a1_edge_index, a1_gid,
           a2_edge_index, a2_gid, b1_edge_index, b1_gid,
           b2_edge_index, b2_gid):
    p = params
    sets = [
        (solu_node, solu_eattr, solu_edge_index, solu_gid, p['gnn_solu']),
        (a1_node, a1_eattr, a1_edge_index, a1_gid, p['gnn_solv_a']),
        (a2_node, a2_eattr, a2_edge_index, a2_gid, p['gnn_solv_a']),
        (b1_node, b1_eattr, b1_edge_index, b1_gid, p['gnn_solv_b']),
        (b2_node, b2_eattr, b2_edge_index, b2_gid, p['gnn_solv_b']),
    ]
    qstars = []
    for node, eattr, ei, gid, gp in sets:
        ew = _edge_net(eattr, gp)
        src = ei[0].reshape(N_EBLK, 1, EB)
        dst = ei[1].reshape(N_EBLK, 1, EB)
        h = _mpnn(node, src, dst, ew, gp)
        qstars.append(_s2s(h, gid.reshape(1, 1, N_NODES), p['s2s']))
    return _head(qstars[0], qstars[1], qstars[2], qstars[3], qstars[4],
                 facs_a, facs_b, temp_a.reshape(-1, 1), temp_b.reshape(-1, 1),
                 p)
